# Initial kernel scaffold; baseline (speedup 1.0000x reference)
import functools

import jax
import jax.numpy as jnp
from jax import lax
from jax.experimental import pallas as pl
from jax.experimental.pallas import tpu as pltpu
from jax.experimental.pallas import tpu_sc as plsc

F32 = jnp.float32
NSUB = 16
LANES = 16
CHUNK = 128
SUB = 128


def _sc_mesh():
    return plsc.VectorSubcoreMesh(
        core_axis_name="c", subcore_axis_name="s", num_cores=1)


def _make_deg_kernel(NP, n_chunks):
    RT = NP // NSUB

    @functools.partial(
        pl.kernel,
        out_type=jax.ShapeDtypeStruct((NP, LANES), F32),
        mesh=_sc_mesh(),
        scratch_types=[
            pltpu.VMEM_SHARED((NP, LANES), F32),
            pltpu.VMEM((n_chunks, CHUNK), jnp.int32),
            pltpu.VMEM((CHUNK, LANES), F32),
            pltpu.VMEM((RT, LANES), F32),
        ],
    )
    def deg_kernel(dst_hbm, deg_out, acc_sh, dst_v, ones_v, zero_v):
        sid = lax.axis_index("s")

        def fill_ones(i, _):
            ones_v[i, :] = jnp.ones((LANES,), F32)
            return 0
        lax.fori_loop(0, CHUNK, fill_ones, 0)

        def fill_zero(i, _):
            zero_v[i, :] = jnp.zeros((LANES,), F32)
            return 0
        lax.fori_loop(0, RT, fill_zero, 0)

        pltpu.sync_copy(dst_hbm.at[pl.ds(sid * n_chunks, n_chunks)], dst_v)
        pltpu.sync_copy(zero_v, acc_sh.at[pl.ds(sid * RT, RT)])
        plsc.subcore_barrier()

        def scat(j, _):
            pltpu.sync_copy(ones_v, acc_sh.at[dst_v.at[j]], add=True)
            return 0
        lax.fori_loop(0, n_chunks, scat, 0)
        plsc.subcore_barrier()

        pltpu.sync_copy(acc_sh.at[pl.ds(sid * RT, RT)],
                        deg_out.at[pl.ds(sid * RT, RT)])

    return deg_kernel


def _dense_body(deg_ref, x_ref, y_ref, dw_ref, w0_ref, w1_ref, lw_ref,
                fw_ref, fb_ref, ufp_ref, ulp_ref, zdw_ref, d1_ref, d2_ref,
                *, DH, DOUT, NLAYERS):
    dot = functools.partial(jnp.dot, precision=lax.Precision.HIGHEST,
                            preferred_element_type=F32)
    deg = deg_ref[...][:, 0:1] + 1.0
    dinv = lax.rsqrt(deg)
    d1_ref[...] = jnp.broadcast_to(dinv, d1_ref.shape)
    d2_ref[...] = jnp.broadcast_to(dinv * dinv, d2_ref.shape)
    fw = fw_ref[...]
    h = dot(x_ref[...], w0_ref[...])
    h = dot(h, w1_ref[...])
    h = dot(h, fw[0:DH])
    ufp_ref[...] = dinv * h
    t = y_ref[...]
    for i in range(NLAYERS):
        t = dot(t, lw_ref[i])
    t = dot(t, fw[DH:DH + DOUT])
    ulp_ref[...] = dinv * t
    zdw_ref[...] = dot(dw_ref[...], fw[DH + DOUT:]) + fb_ref[...]


def _make_dense_kernel(NP, DIN, DH, DOUT, DWD, NLAYERS):
    BR = 512
    grid = (NP // BR,)
    row_blk = lambda w: pl.BlockSpec((BR, w), lambda i: (i, 0))
    full = lambda *shape: pl.BlockSpec(shape, lambda i: tuple(0 for _ in shape))
    out_sdt = jax.ShapeDtypeStruct((NP, DOUT), F32)
    return pl.pallas_call(
        functools.partial(_dense_body, DH=DH, DOUT=DOUT, NLAYERS=NLAYERS),
        grid=grid,
        in_specs=[
            row_blk(LANES), row_blk(DIN), row_blk(DOUT), row_blk(DWD),
            full(DIN, DH), full(DH, DH), full(NLAYERS, DOUT, DOUT),
            full(DH + DOUT + DWD, DOUT), full(1, DOUT),
        ],
        out_specs=[row_blk(DOUT)] * 5,
        out_shape=[out_sdt] * 5,
    )


def _make_chain_kernel(NP, DOUT, n_chunks, n_fp, n_lp):
    RT = NP // NSUB
    NSC = RT // SUB
    VPR = DOUT // LANES

    @functools.partial(
        pl.kernel,
        out_type=[
            jax.ShapeDtypeStruct((NP, DOUT), F32),
            jax.ShapeDtypeStruct((NP, DOUT), F32),
            jax.ShapeDtypeStruct((NP, DOUT), F32),
        ],
        mesh=_sc_mesh(),
        scratch_types=[
            pltpu.VMEM_SHARED((NP, DOUT), F32),
            pltpu.VMEM((n_chunks, CHUNK), jnp.int32),
            pltpu.VMEM((n_chunks, CHUNK), jnp.int32),
            pltpu.VMEM((CHUNK, DOUT), F32),
            pltpu.VMEM((SUB, DOUT), F32),
            pltpu.VMEM((SUB, DOUT), F32),
            pltpu.VMEM((SUB, DOUT), F32),
            pltpu.VMEM((SUB, DOUT), F32),
            pltpu.VMEM((SUB, DOUT), F32),
            pltpu.VMEM((16, DOUT), F32),
            pltpu.SemaphoreType.DMA,
        ],
    )
    def chain_kernel(src_hbm, dst_hbm, cmat_hbm, ufp0_hbm, ulp0_hbm,
                     zdw_hbm, d1_hbm, d2_hbm, out_hbm, ufp_hbm, ulp_hbm,
                     acc_sh, src_v, dst_v, rows_v, a_v, u_v, s1_v, s2_v,
                     zero_v, cm_v, sem):
        sid = lax.axis_index("s")
        base_r = sid * RT
        base_c = sid * n_chunks

        pltpu.sync_copy(src_hbm.at[pl.ds(base_c, n_chunks)], src_v)
        pltpu.sync_copy(dst_hbm.at[pl.ds(base_c, n_chunks)], dst_v)
        pltpu.sync_copy(cmat_hbm, cm_v)

        def fill_zero(i, _):
            for j in range(VPR):
                zero_v[i, pl.ds(j * LANES, LANES)] = jnp.zeros((LANES,), F32)
            return 0
        lax.fori_loop(0, SUB, fill_zero, 0)

        for i in range(NSC):
            sl = pl.ds(base_r + i * SUB, SUB)
            pltpu.sync_copy(ufp0_hbm.at[sl], u_v)
            pltpu.sync_copy(u_v, ufp_hbm.at[sl])
            pltpu.sync_copy(ulp0_hbm.at[sl], u_v)
            pltpu.sync_copy(u_v, ulp_hbm.at[sl])
        plsc.subcore_barrier()

        def do_step(u_hbm, crow, last):
            for i in range(NSC):
                pltpu.sync_copy(zero_v, acc_sh.at[pl.ds(base_r + i * SUB, SUB)])
            plsc.subcore_barrier()

            def edge_body(j, _):
                pltpu.async_copy(u_hbm.at[src_v.at[j]], rows_v, sem).wait()
                pltpu.sync_copy(rows_v, acc_sh.at[dst_v.at[j]], add=True)
                return 0
            lax.fori_loop(0, n_chunks, edge_body, 0)
            plsc.subcore_barrier()

            for i in range(NSC):
                sl = pl.ds(base_r + i * SUB, SUB)
                pltpu.sync_copy(acc_sh.at[sl], a_v)
                pltpu.sync_copy(u_hbm.at[sl], u_v)
                if last:
                    pltpu.sync_copy(d1_hbm.at[sl], s2_v)
                else:
                    pltpu.sync_copy(d2_hbm.at[sl], s2_v)
                    pltpu.sync_copy(d1_hbm.at[sl], s1_v)

                def row_body(r, _):
                    for j in range(VPR):
                        cs = pl.ds(j * LANES, LANES)
                        t = (a_v[r, cs] + u_v[r, cs]) * s2_v[r, cs]
                        cv = cm_v[crow, cs]
                        if last:
                            t = t + cv
                        else:
                            t = t + s1_v[r, cs] * cv
                        u_v[r, cs] = t
                    return 0
                lax.fori_loop(0, SUB, row_body, 0)
                pltpu.sync_copy(u_v, u_hbm.at[sl])
            plsc.subcore_barrier()

        for k in range(n_fp):
            do_step(ufp_hbm, k, k == n_fp - 1)
        for k in range(n_lp):
            do_step(ulp_hbm, n_fp + k, k == n_lp - 1)

        for i in range(NSC):
            sl = pl.ds(base_r + i * SUB, SUB)
            pltpu.sync_copy(ufp_hbm.at[sl], a_v)
            pltpu.sync_copy(ulp_hbm.at[sl], u_v)
            pltpu.sync_copy(zdw_hbm.at[sl], s1_v)

            def fin_body(r, _):
                for j in range(VPR):
                    cs = pl.ds(j * LANES, LANES)
                    t = a_v[r, cs] + u_v[r, cs] + s1_v[r, cs]
                    a_v[r, cs] = 1.0 / (1.0 + jnp.exp(-t))
                return 0
            lax.fori_loop(0, SUB, fin_body, 0)
            pltpu.sync_copy(a_v, out_hbm.at[sl])

    return chain_kernel


def kernel(x, y, edge_index, deep_walk_emb, gcn_W0, gcn_b0, gcn_W1, gcn_b1,
           label_W, label_b, fus_W, fus_b):
    N, DIN = x.shape
    DOUT = y.shape[1]
    DWD = deep_walk_emb.shape[1]
    DH = gcn_W0.shape[1]
    NLAYERS = label_W.shape[0]
    E = edge_index.shape[1]

    NP = ((N + NSUB * SUB - 1) // (NSUB * SUB)) * (NSUB * SUB)
    n_chunks = (E + NSUB * CHUNK - 1) // (NSUB * CHUNK)
    EP = n_chunks * NSUB * CHUNK

    src = edge_index[0]
    dst = edge_index[1]
    pad_e = EP - E
    src_p = jnp.concatenate([src, jnp.zeros((pad_e,), jnp.int32)])
    dst_p = jnp.concatenate([dst, jnp.full((pad_e,), N, jnp.int32)])
    src2 = src_p.reshape(NSUB * n_chunks, CHUNK)
    dst2 = dst_p.reshape(NSUB * n_chunks, CHUNK)

    pad_rows = lambda a: jnp.pad(a, ((0, NP - N), (0, 0)))
    x_p = pad_rows(x)
    y_p = pad_rows(y)
    dw_p = pad_rows(deep_walk_emb)

    Wf_fp = fus_W[:DH]
    Wf_lp = fus_W[DH:DH + DOUT]
    c_fp0 = gcn_b0 @ gcn_W1 @ Wf_fp
    c_fp1 = gcn_b1 @ Wf_fp
    M = Wf_lp
    cs = [None] * NLAYERS
    for k in range(NLAYERS - 1, -1, -1):
        cs[k] = label_b[k] @ M
        M = label_W[k] @ M
    cmat = jnp.zeros((16, DOUT), F32)
    cmat = cmat.at[0].set(c_fp0).at[1].set(c_fp1)
    for k in range(NLAYERS):
        cmat = cmat.at[2 + k].set(cs[k])

    deg_raw = _make_deg_kernel(NP, n_chunks)(dst2)

    ufp0, ulp0, zdwb, d1, d2 = _make_dense_kernel(
        NP, DIN, DH, DOUT, DWD, NLAYERS)(
        deg_raw, x_p, y_p, dw_p, gcn_W0, gcn_W1, label_W, fus_W,
        fus_b.reshape(1, DOUT))

    out, _, _ = _make_chain_kernel(NP, DOUT, n_chunks, 2, NLAYERS)(
        src2, dst2, cmat, ufp0, ulp0, zdwb, d1, d2)

    return out[:N]


# speedup vs baseline: 5.3954x; 5.3954x over previous
import functools

import jax
import jax.numpy as jnp
from jax import lax
from jax.experimental import pallas as pl
from jax.experimental.pallas import tpu as pltpu
from jax.experimental.pallas import tpu_sc as plsc

F32 = jnp.float32
NSUB = 16
LANES = 16
CHUNK = 128
SUB = 128


def _sc_mesh():
    return plsc.VectorSubcoreMesh(
        core_axis_name="c", subcore_axis_name="s", num_cores=1)


def _make_deg_kernel(NP, n_chunks):
    RT = NP // NSUB

    @functools.partial(
        pl.kernel,
        out_type=jax.ShapeDtypeStruct((NP, LANES), F32),
        mesh=_sc_mesh(),
        compiler_params=pltpu.CompilerParams(use_tc_tiling_on_sc=False),
        scratch_types=[
            pltpu.VMEM_SHARED((NP, LANES), F32),
            pltpu.VMEM((n_chunks, CHUNK), jnp.int32),
            pltpu.VMEM((CHUNK,), jnp.int32),
            pltpu.VMEM((CHUNK, LANES), F32),
            pltpu.VMEM((RT, LANES), F32),
        ],
    )
    def deg_kernel(dst_hbm, deg_out, acc_sh, dst_v, idx_v, ones_v, zero_v):
        sid = lax.axis_index("s")

        def fill_ones(i, _):
            ones_v[i, :] = jnp.ones((LANES,), F32)
            return 0
        lax.fori_loop(0, CHUNK, fill_ones, 0)

        def fill_zero(i, _):
            zero_v[i, :] = jnp.zeros((LANES,), F32)
            return 0
        lax.fori_loop(0, RT, fill_zero, 0)

        pltpu.sync_copy(dst_hbm.at[pl.ds(sid * n_chunks, n_chunks)], dst_v)
        pltpu.sync_copy(zero_v, acc_sh.at[pl.ds(sid * RT, RT)])
        plsc.subcore_barrier()

        def scat(j, _):
            for i in range(CHUNK // LANES):
                idx_v[pl.ds(i * LANES, LANES)] = dst_v[j, pl.ds(i * LANES, LANES)]
            pltpu.sync_copy(ones_v, acc_sh.at[idx_v], add=True)
            return 0
        lax.fori_loop(0, n_chunks, scat, 0)
        plsc.subcore_barrier()

        pltpu.sync_copy(acc_sh.at[pl.ds(sid * RT, RT)],
                        deg_out.at[pl.ds(sid * RT, RT)])

    return deg_kernel


def _dense_body(deg_ref, x_ref, y_ref, dw_ref, w0_ref, w1_ref, lw_ref,
                fw_ref, fb_ref, ufp_ref, ulp_ref, zdw_ref, d1_ref, d2_ref,
                *, DH, DOUT, NLAYERS):
    dot = functools.partial(jnp.dot, precision=lax.Precision.HIGHEST,
                            preferred_element_type=F32)
    deg = deg_ref[...][:, 0:1] + 1.0
    dinv = lax.rsqrt(deg)
    d1_ref[...] = jnp.broadcast_to(dinv, d1_ref.shape)
    d2_ref[...] = jnp.broadcast_to(dinv * dinv, d2_ref.shape)
    fw = fw_ref[...]
    h = dot(x_ref[...], w0_ref[...])
    h = dot(h, w1_ref[...])
    h = dot(h, fw[0:DH])
    ufp_ref[...] = dinv * h
    t = y_ref[...]
    for i in range(NLAYERS):
        t = dot(t, lw_ref[i])
    t = dot(t, fw[DH:DH + DOUT])
    ulp_ref[...] = dinv * t
    zdw_ref[...] = dot(dw_ref[...], fw[DH + DOUT:]) + fb_ref[...]


def _make_dense_kernel(NP, DIN, DH, DOUT, DWD, NLAYERS):
    BR = 512
    grid = (NP // BR,)
    row_blk = lambda w: pl.BlockSpec((BR, w), lambda i: (i, 0))
    full = lambda *shape: pl.BlockSpec(shape, lambda i: tuple(0 for _ in shape))
    out_sdt = jax.ShapeDtypeStruct((NP, DOUT), F32)
    return pl.pallas_call(
        functools.partial(_dense_body, DH=DH, DOUT=DOUT, NLAYERS=NLAYERS),
        grid=grid,
        in_specs=[
            row_blk(LANES), row_blk(DIN), row_blk(DOUT), row_blk(DWD),
            full(DIN, DH), full(DH, DH), full(NLAYERS, DOUT, DOUT),
            full(DH + DOUT + DWD, DOUT), full(1, DOUT),
        ],
        out_specs=[row_blk(DOUT)] * 5,
        out_shape=[out_sdt] * 5,
    )


def _make_chain_kernel(NP, DOUT, n_chunks, n_fp, n_lp):
    RT = NP // NSUB
    NSC = RT // SUB
    VPR = DOUT // LANES
    ZR = 64

    @functools.partial(
        pl.kernel,
        out_type=[
            jax.ShapeDtypeStruct((NP, DOUT), F32),
            jax.ShapeDtypeStruct((NP, DOUT), F32),
            jax.ShapeDtypeStruct((NP, DOUT), F32),
        ],
        mesh=_sc_mesh(),
        compiler_params=pltpu.CompilerParams(use_tc_tiling_on_sc=False),
        scratch_types=[
            pltpu.VMEM_SHARED((NP, DOUT), F32),
            pltpu.VMEM((n_chunks, CHUNK), jnp.int32),
            pltpu.VMEM((n_chunks, CHUNK), jnp.int32),
            pltpu.VMEM((CHUNK,), jnp.int32),
            pltpu.VMEM((CHUNK, DOUT), F32),
            pltpu.VMEM((SUB, DOUT), F32),
            pltpu.VMEM((SUB, DOUT), F32),
            pltpu.VMEM((SUB, DOUT), F32),
            pltpu.VMEM((SUB, DOUT), F32),
            pltpu.VMEM((ZR, DOUT), F32),
            pltpu.VMEM((16, DOUT), F32),
            pltpu.SemaphoreType.DMA,
        ],
    )
    def chain_kernel(src_hbm, dst_hbm, cmat_hbm, ufp0_hbm, ulp0_hbm,
                     zdw_hbm, d1_hbm, d2_hbm, out_hbm, ufp_hbm, ulp_hbm,
                     acc_sh, src_v, dst_v, idx_v, rows_v, a_v, u_v, s1_v,
                     s2_v, zero_v, cm_v, sem):
        sid = lax.axis_index("s")
        base_r = sid * RT
        base_c = sid * n_chunks

        pltpu.sync_copy(src_hbm.at[pl.ds(base_c, n_chunks)], src_v)
        pltpu.sync_copy(dst_hbm.at[pl.ds(base_c, n_chunks)], dst_v)
        pltpu.sync_copy(cmat_hbm, cm_v)

        def fill_zero(i, _):
            for j in range(VPR):
                zero_v[i, pl.ds(j * LANES, LANES)] = jnp.zeros((LANES,), F32)
            return 0
        lax.fori_loop(0, ZR, fill_zero, 0)

        for i in range(NSC):
            sl = pl.ds(base_r + i * SUB, SUB)
            pltpu.sync_copy(ufp0_hbm.at[sl], u_v)
            pltpu.sync_copy(u_v, ufp_hbm.at[sl])
            pltpu.sync_copy(ulp0_hbm.at[sl], u_v)
            pltpu.sync_copy(u_v, ulp_hbm.at[sl])
        plsc.subcore_barrier()

        def do_step(u_hbm, crow, last):
            for i in range(RT // ZR):
                pltpu.sync_copy(zero_v, acc_sh.at[pl.ds(base_r + i * ZR, ZR)])
            plsc.subcore_barrier()

            def edge_body(j, _):
                pltpu.async_copy(u_hbm.at[src_v.at[j]], rows_v, sem).wait()
                for i in range(CHUNK // LANES):
                    idx_v[pl.ds(i * LANES, LANES)] = (
                        dst_v[j, pl.ds(i * LANES, LANES)])
                pltpu.sync_copy(rows_v, acc_sh.at[idx_v], add=True)
                return 0
            lax.fori_loop(0, n_chunks, edge_body, 0)
            plsc.subcore_barrier()

            for i in range(NSC):
                sl = pl.ds(base_r + i * SUB, SUB)
                pltpu.sync_copy(acc_sh.at[sl], a_v)
                pltpu.sync_copy(u_hbm.at[sl], u_v)
                if last:
                    pltpu.sync_copy(d1_hbm.at[sl], s2_v)
                else:
                    pltpu.sync_copy(d2_hbm.at[sl], s2_v)
                    pltpu.sync_copy(d1_hbm.at[sl], s1_v)

                def row_body(r, _):
                    for j in range(VPR):
                        cs = pl.ds(j * LANES, LANES)
                        t = (a_v[r, cs] + u_v[r, cs]) * s2_v[r, cs]
                        cv = cm_v[crow, cs]
                        if last:
                            t = t + cv
                        else:
                            t = t + s1_v[r, cs] * cv
                        u_v[r, cs] = t
                    return 0
                lax.fori_loop(0, SUB, row_body, 0)
                pltpu.sync_copy(u_v, u_hbm.at[sl])
            plsc.subcore_barrier()

        for k in range(n_fp - 1):
            do_step(ufp_hbm, k, False)
        do_step(ufp_hbm, n_fp - 1, True)

        def lp_body(k, _):
            do_step(ulp_hbm, n_fp + k, False)
            return 0
        lax.fori_loop(0, n_lp - 1, lp_body, 0)
        do_step(ulp_hbm, n_fp + n_lp - 1, True)

        for i in range(NSC):
            sl = pl.ds(base_r + i * SUB, SUB)
            pltpu.sync_copy(ufp_hbm.at[sl], a_v)
            pltpu.sync_copy(ulp_hbm.at[sl], u_v)
            pltpu.sync_copy(zdw_hbm.at[sl], s1_v)

            def fin_body(r, _):
                for j in range(VPR):
                    cs = pl.ds(j * LANES, LANES)
                    t = a_v[r, cs] + u_v[r, cs] + s1_v[r, cs]
                    a_v[r, cs] = 1.0 / (1.0 + jnp.exp(-t))
                return 0
            lax.fori_loop(0, SUB, fin_body, 0)
            pltpu.sync_copy(a_v, out_hbm.at[sl])

    return chain_kernel


def kernel(x, y, edge_index, deep_walk_emb, gcn_W0, gcn_b0, gcn_W1, gcn_b1,
           label_W, label_b, fus_W, fus_b):
    N, DIN = x.shape
    DOUT = y.shape[1]
    DWD = deep_walk_emb.shape[1]
    DH = gcn_W0.shape[1]
    NLAYERS = label_W.shape[0]
    E = edge_index.shape[1]

    NP = ((N + NSUB * SUB - 1) // (NSUB * SUB)) * (NSUB * SUB)
    n_chunks = (E + NSUB * CHUNK - 1) // (NSUB * CHUNK)
    n_chunks = ((n_chunks + 7) // 8) * 8
    EP = n_chunks * NSUB * CHUNK

    src = edge_index[0]
    dst = edge_index[1]
    pad_e = EP - E
    src_p = jnp.concatenate([src, jnp.zeros((pad_e,), jnp.int32)])
    dst_p = jnp.concatenate([dst, jnp.full((pad_e,), N, jnp.int32)])
    src2 = src_p.reshape(NSUB * n_chunks, CHUNK)
    dst2 = dst_p.reshape(NSUB * n_chunks, CHUNK)

    pad_rows = lambda a: jnp.pad(a, ((0, NP - N), (0, 0)))
    x_p = pad_rows(x)
    y_p = pad_rows(y)
    dw_p = pad_rows(deep_walk_emb)

    Wf_fp = fus_W[:DH]
    Wf_lp = fus_W[DH:DH + DOUT]
    c_fp0 = gcn_b0 @ gcn_W1 @ Wf_fp
    c_fp1 = gcn_b1 @ Wf_fp
    M = Wf_lp
    cs = [None] * NLAYERS
    for k in range(NLAYERS - 1, -1, -1):
        cs[k] = label_b[k] @ M
        M = label_W[k] @ M
    cmat = jnp.zeros((16, DOUT), F32)
    cmat = cmat.at[0].set(c_fp0).at[1].set(c_fp1)
    for k in range(NLAYERS):
        cmat = cmat.at[2 + k].set(cs[k])

    deg_raw = _make_deg_kernel(NP, n_chunks)(dst2)

    ufp0, ulp0, zdwb, d1, d2 = _make_dense_kernel(
        NP, DIN, DH, DOUT, DWD, NLAYERS)(
        deg_raw, x_p, y_p, dw_p, gcn_W0, gcn_W1, label_W, fus_W,
        fus_b.reshape(1, DOUT))

    out, _, _ = _make_chain_kernel(NP, DOUT, n_chunks, 2, NLAYERS)(
        src2, dst2, cmat, ufp0, ulp0, zdwb, d1, d2)

    return out[:N]


# speedup vs baseline: 6.9557x; 1.2892x over previous
import functools

import jax
import jax.numpy as jnp
from jax import lax
from jax.experimental import pallas as pl
from jax.experimental.pallas import tpu as pltpu
from jax.experimental.pallas import tpu_sc as plsc

F32 = jnp.float32
NSUB = 16
LANES = 16
CHUNK = 128
SUB = 128


def _sc_mesh():
    return plsc.VectorSubcoreMesh(
        core_axis_name="c", subcore_axis_name="s", num_cores=1)


def _make_deg_kernel(NP, n_chunks):
    RT = NP // NSUB

    @functools.partial(
        pl.kernel,
        out_type=jax.ShapeDtypeStruct((NP, LANES), F32),
        mesh=_sc_mesh(),
        compiler_params=pltpu.CompilerParams(use_tc_tiling_on_sc=False),
        scratch_types=[
            pltpu.VMEM_SHARED((NP, LANES), F32),
            pltpu.VMEM((n_chunks, CHUNK), jnp.int32),
            pltpu.VMEM((CHUNK,), jnp.int32),
            pltpu.VMEM((CHUNK, LANES), F32),
            pltpu.VMEM((RT, LANES), F32),
        ],
    )
    def deg_kernel(dst_hbm, deg_out, acc_sh, dst_v, idx_v, ones_v, zero_v):
        sid = lax.axis_index("s")

        def fill_ones(i, _):
            ones_v[i, :] = jnp.ones((LANES,), F32)
            return 0
        lax.fori_loop(0, CHUNK, fill_ones, 0)

        def fill_zero(i, _):
            zero_v[i, :] = jnp.zeros((LANES,), F32)
            return 0
        lax.fori_loop(0, RT, fill_zero, 0)

        pltpu.sync_copy(dst_hbm.at[pl.ds(sid * n_chunks, n_chunks)], dst_v)
        pltpu.sync_copy(zero_v, acc_sh.at[pl.ds(sid * RT, RT)])
        plsc.subcore_barrier()

        def scat(j, _):
            for i in range(CHUNK // LANES):
                idx_v[pl.ds(i * LANES, LANES)] = dst_v[j, pl.ds(i * LANES, LANES)]
            pltpu.sync_copy(ones_v, acc_sh.at[idx_v], add=True)
            return 0
        lax.fori_loop(0, n_chunks, scat, 0)
        plsc.subcore_barrier()

        pltpu.sync_copy(acc_sh.at[pl.ds(sid * RT, RT)],
                        deg_out.at[pl.ds(sid * RT, RT)])

    return deg_kernel


def _dense_body(deg_ref, x_ref, y_ref, dw_ref, w0_ref, w1_ref, lw_ref,
                fw_ref, fb_ref, ufp_ref, ulp_ref, zdw_ref, d1_ref, d2_ref,
                *, DH, DOUT, NLAYERS):
    dot = functools.partial(jnp.dot, precision=lax.Precision.HIGHEST,
                            preferred_element_type=F32)
    deg = deg_ref[...][:, 0:1] + 1.0
    dinv = lax.rsqrt(deg)
    d1_ref[...] = jnp.broadcast_to(dinv, d1_ref.shape)
    d2_ref[...] = jnp.broadcast_to(dinv * dinv, d2_ref.shape)
    fw = fw_ref[...]
    h = dot(x_ref[...], w0_ref[...])
    h = dot(h, w1_ref[...])
    h = dot(h, fw[0:DH])
    ufp_ref[...] = dinv * h
    t = y_ref[...]
    for i in range(NLAYERS):
        t = dot(t, lw_ref[i])
    t = dot(t, fw[DH:DH + DOUT])
    ulp_ref[...] = dinv * t
    zdw_ref[...] = dot(dw_ref[...], fw[DH + DOUT:]) + fb_ref[...]


def _make_dense_kernel(NP, DIN, DH, DOUT, DWD, NLAYERS):
    BR = 512
    grid = (NP // BR,)
    row_blk = lambda w: pl.BlockSpec((BR, w), lambda i: (i, 0))
    full = lambda *shape: pl.BlockSpec(shape, lambda i: tuple(0 for _ in shape))
    out_sdt = jax.ShapeDtypeStruct((NP, DOUT), F32)
    return pl.pallas_call(
        functools.partial(_dense_body, DH=DH, DOUT=DOUT, NLAYERS=NLAYERS),
        grid=grid,
        in_specs=[
            row_blk(LANES), row_blk(DIN), row_blk(DOUT), row_blk(DWD),
            full(DIN, DH), full(DH, DH), full(NLAYERS, DOUT, DOUT),
            full(DH + DOUT + DWD, DOUT), full(1, DOUT),
        ],
        out_specs=[row_blk(DOUT)] * 5,
        out_shape=[out_sdt] * 5,
    )


def _make_chain_kernel(NP, DOUT, n_chunks, n_fp, n_lp):
    RT = NP // NSUB
    NSC = RT // SUB
    VPR = DOUT // LANES
    ZR = 64

    @functools.partial(
        pl.kernel,
        out_type=[
            jax.ShapeDtypeStruct((NP, DOUT), F32),
            jax.ShapeDtypeStruct((NP, DOUT), F32),
            jax.ShapeDtypeStruct((NP, DOUT), F32),
        ],
        mesh=_sc_mesh(),
        compiler_params=pltpu.CompilerParams(use_tc_tiling_on_sc=False),
        scratch_types=[
            pltpu.VMEM_SHARED((NP, DOUT), F32),
            pltpu.VMEM((n_chunks, CHUNK), jnp.int32),
            pltpu.VMEM((n_chunks, CHUNK), jnp.int32),
            [pltpu.VMEM((CHUNK,), jnp.int32)] * 4,
            pltpu.VMEM((SUB, DOUT), F32),
            pltpu.VMEM((SUB, DOUT), F32),
            pltpu.VMEM((SUB, DOUT), F32),
            pltpu.VMEM((SUB, DOUT), F32),
            pltpu.VMEM((ZR, DOUT), F32),
            pltpu.VMEM((16, DOUT), F32),
            pltpu.SemaphoreType.DMA,
            pltpu.SemaphoreType.DMA,
        ],
    )
    def chain_kernel(src_hbm, dst_hbm, cmat_hbm, ufp0_hbm, ulp0_hbm,
                     zdw_hbm, d1_hbm, d2_hbm, out_hbm, ufp_hbm, ulp_hbm,
                     acc_sh, src_v, dst_v, idxr_v, a_v, u_v, s1_v,
                     s2_v, zero_v, cm_v, gsem, ssem):
        sid = lax.axis_index("s")
        base_r = sid * RT
        base_c = sid * n_chunks

        pltpu.sync_copy(src_hbm.at[pl.ds(base_c, n_chunks)], src_v)
        pltpu.sync_copy(dst_hbm.at[pl.ds(base_c, n_chunks)], dst_v)
        pltpu.sync_copy(cmat_hbm, cm_v)

        def fill_zero(i, _):
            for j in range(VPR):
                zero_v[i, pl.ds(j * LANES, LANES)] = jnp.zeros((LANES,), F32)
            return 0
        lax.fori_loop(0, ZR, fill_zero, 0)

        for i in range(NSC):
            sl = pl.ds(base_r + i * SUB, SUB)
            pltpu.sync_copy(ufp0_hbm.at[sl], u_v)
            pltpu.sync_copy(u_v, ufp_hbm.at[sl])
            pltpu.sync_copy(ulp0_hbm.at[sl], u_v)
            pltpu.sync_copy(u_v, ulp_hbm.at[sl])
        plsc.subcore_barrier()

        def do_step(u_hbm, crow, last):
            for i in range(RT // ZR):
                pltpu.sync_copy(zero_v, acc_sh.at[pl.ds(base_r + i * ZR, ZR)])
            plsc.subcore_barrier()

            bufs = (a_v, u_v, s1_v, s2_v)

            def edge_group(g, _):
                c0 = g * 4
                for b in range(4):
                    pltpu.async_copy(u_hbm.at[src_v.at[c0 + b]], bufs[b],
                                     gsem)
                for b in range(4):
                    pltpu.make_async_copy(u_hbm.at[src_v.at[c0 + b]],
                                          bufs[b], gsem).wait()
                    for i in range(CHUNK // LANES):
                        idxr_v[b][pl.ds(i * LANES, LANES)] = (
                            dst_v[c0 + b, pl.ds(i * LANES, LANES)])
                    pltpu.async_copy(bufs[b], acc_sh.at[idxr_v[b]],
                                     ssem, add=True)
                for b in range(4):
                    pltpu.make_async_copy(bufs[b], acc_sh.at[idxr_v[b]],
                                          ssem).wait()
                return 0
            lax.fori_loop(0, n_chunks // 4, edge_group, 0)
            plsc.subcore_barrier()

            for i in range(NSC):
                sl = pl.ds(base_r + i * SUB, SUB)
                pltpu.sync_copy(acc_sh.at[sl], a_v)
                pltpu.sync_copy(u_hbm.at[sl], u_v)
                if last:
                    pltpu.sync_copy(d1_hbm.at[sl], s2_v)
                else:
                    pltpu.sync_copy(d2_hbm.at[sl], s2_v)
                    pltpu.sync_copy(d1_hbm.at[sl], s1_v)

                def row_body(r, _):
                    for j in range(VPR):
                        cs = pl.ds(j * LANES, LANES)
                        t = (a_v[r, cs] + u_v[r, cs]) * s2_v[r, cs]
                        cv = cm_v[crow, cs]
                        if last:
                            t = t + cv
                        else:
                            t = t + s1_v[r, cs] * cv
                        u_v[r, cs] = t
                    return 0
                lax.fori_loop(0, SUB, row_body, 0)
                pltpu.sync_copy(u_v, u_hbm.at[sl])
            plsc.subcore_barrier()

        for k in range(n_fp - 1):
            do_step(ufp_hbm, k, False)
        do_step(ufp_hbm, n_fp - 1, True)

        def lp_body(k, _):
            do_step(ulp_hbm, n_fp + k, False)
            return 0
        lax.fori_loop(0, n_lp - 1, lp_body, 0)
        do_step(ulp_hbm, n_fp + n_lp - 1, True)

        for i in range(NSC):
            sl = pl.ds(base_r + i * SUB, SUB)
            pltpu.sync_copy(ufp_hbm.at[sl], a_v)
            pltpu.sync_copy(ulp_hbm.at[sl], u_v)
            pltpu.sync_copy(zdw_hbm.at[sl], s1_v)

            def fin_body(r, _):
                for j in range(VPR):
                    cs = pl.ds(j * LANES, LANES)
                    t = a_v[r, cs] + u_v[r, cs] + s1_v[r, cs]
                    a_v[r, cs] = 1.0 / (1.0 + jnp.exp(-t))
                return 0
            lax.fori_loop(0, SUB, fin_body, 0)
            pltpu.sync_copy(a_v, out_hbm.at[sl])

    return chain_kernel


def kernel(x, y, edge_index, deep_walk_emb, gcn_W0, gcn_b0, gcn_W1, gcn_b1,
           label_W, label_b, fus_W, fus_b):
    N, DIN = x.shape
    DOUT = y.shape[1]
    DWD = deep_walk_emb.shape[1]
    DH = gcn_W0.shape[1]
    NLAYERS = label_W.shape[0]
    E = edge_index.shape[1]

    NP = ((N + NSUB * SUB - 1) // (NSUB * SUB)) * (NSUB * SUB)
    n_chunks = (E + NSUB * CHUNK - 1) // (NSUB * CHUNK)
    n_chunks = ((n_chunks + 7) // 8) * 8
    EP = n_chunks * NSUB * CHUNK

    src = edge_index[0]
    dst = edge_index[1]
    pad_e = EP - E
    src_p = jnp.concatenate([src, jnp.zeros((pad_e,), jnp.int32)])
    dst_p = jnp.concatenate([dst, jnp.full((pad_e,), N, jnp.int32)])
    src2 = src_p.reshape(NSUB * n_chunks, CHUNK)
    dst2 = dst_p.reshape(NSUB * n_chunks, CHUNK)

    pad_rows = lambda a: jnp.pad(a, ((0, NP - N), (0, 0)))
    x_p = pad_rows(x)
    y_p = pad_rows(y)
    dw_p = pad_rows(deep_walk_emb)

    Wf_fp = fus_W[:DH]
    Wf_lp = fus_W[DH:DH + DOUT]
    c_fp0 = gcn_b0 @ gcn_W1 @ Wf_fp
    c_fp1 = gcn_b1 @ Wf_fp
    M = Wf_lp
    cs = [None] * NLAYERS
    for k in range(NLAYERS - 1, -1, -1):
        cs[k] = label_b[k] @ M
        M = label_W[k] @ M
    cmat = jnp.zeros((16, DOUT), F32)
    cmat = cmat.at[0].set(c_fp0).at[1].set(c_fp1)
    for k in range(NLAYERS):
        cmat = cmat.at[2 + k].set(cs[k])

    deg_raw = _make_deg_kernel(NP, n_chunks)(dst2)

    ufp0, ulp0, zdwb, d1, d2 = _make_dense_kernel(
        NP, DIN, DH, DOUT, DWD, NLAYERS)(
        deg_raw, x_p, y_p, dw_p, gcn_W0, gcn_W1, label_W, fus_W,
        fus_b.reshape(1, DOUT))

    out, _, _ = _make_chain_kernel(NP, DOUT, n_chunks, 2, NLAYERS)(
        src2, dst2, cmat, ufp0, ulp0, zdwb, d1, d2)

    return out[:N]


# speedup vs baseline: 7.2237x; 1.0385x over previous
import functools

import jax
import jax.numpy as jnp
from jax import lax
from jax.experimental import pallas as pl
from jax.experimental.pallas import tpu as pltpu
from jax.experimental.pallas import tpu_sc as plsc

F32 = jnp.float32
NSUB = 16
LANES = 16
CHUNK = 128
SUB = 128


def _sc_mesh():
    return plsc.VectorSubcoreMesh(
        core_axis_name="c", subcore_axis_name="s", num_cores=1)


def _make_deg_kernel(NP, n_chunks):
    RT = NP // NSUB

    @functools.partial(
        pl.kernel,
        out_type=jax.ShapeDtypeStruct((NP, LANES), F32),
        mesh=_sc_mesh(),
        compiler_params=pltpu.CompilerParams(use_tc_tiling_on_sc=False),
        scratch_types=[
            pltpu.VMEM_SHARED((NP, LANES), F32),
            pltpu.VMEM((n_chunks, CHUNK), jnp.int32),
            pltpu.VMEM((CHUNK,), jnp.int32),
            pltpu.VMEM((CHUNK, LANES), F32),
            pltpu.VMEM((RT, LANES), F32),
        ],
    )
    def deg_kernel(dst_hbm, deg_out, acc_sh, dst_v, idx_v, ones_v, zero_v):
        sid = lax.axis_index("s")

        def fill_ones(i, _):
            ones_v[i, :] = jnp.ones((LANES,), F32)
            return 0
        lax.fori_loop(0, CHUNK, fill_ones, 0)

        def fill_zero(i, _):
            zero_v[i, :] = jnp.zeros((LANES,), F32)
            return 0
        lax.fori_loop(0, RT, fill_zero, 0)

        pltpu.sync_copy(dst_hbm.at[pl.ds(sid * n_chunks, n_chunks)], dst_v)
        pltpu.sync_copy(zero_v, acc_sh.at[pl.ds(sid * RT, RT)])
        plsc.subcore_barrier()

        def scat(j, _):
            for i in range(CHUNK // LANES):
                idx_v[pl.ds(i * LANES, LANES)] = dst_v[j, pl.ds(i * LANES, LANES)]
            pltpu.sync_copy(ones_v, acc_sh.at[idx_v], add=True)
            return 0
        lax.fori_loop(0, n_chunks, scat, 0)
        plsc.subcore_barrier()

        pltpu.sync_copy(acc_sh.at[pl.ds(sid * RT, RT)],
                        deg_out.at[pl.ds(sid * RT, RT)])

    return deg_kernel


def _dense_body(deg_ref, x_ref, y_ref, dw_ref, w0_ref, w1_ref, lw_ref,
                fw_ref, fb_ref, ufp_ref, ulp_ref, zdw_ref, d1_ref, d2_ref,
                *, DH, DOUT, NLAYERS):
    dot = functools.partial(jnp.dot, precision=lax.Precision.HIGHEST,
                            preferred_element_type=F32)
    deg = deg_ref[...][:, 0:1] + 1.0
    dinv = lax.rsqrt(deg)
    d1_ref[...] = jnp.broadcast_to(dinv, d1_ref.shape)
    d2_ref[...] = jnp.broadcast_to(dinv * dinv, d2_ref.shape)
    fw = fw_ref[...]
    h = dot(x_ref[...], w0_ref[...])
    h = dot(h, w1_ref[...])
    h = dot(h, fw[0:DH])
    ufp_ref[...] = dinv * h
    t = y_ref[...]
    for i in range(NLAYERS):
        t = dot(t, lw_ref[i])
    t = dot(t, fw[DH:DH + DOUT])
    ulp_ref[...] = dinv * t
    zdw_ref[...] = dot(dw_ref[...], fw[DH + DOUT:]) + fb_ref[...]


def _make_dense_kernel(NP, DIN, DH, DOUT, DWD, NLAYERS):
    BR = 512
    grid = (NP // BR,)
    row_blk = lambda w: pl.BlockSpec((BR, w), lambda i: (i, 0))
    full = lambda *shape: pl.BlockSpec(shape, lambda i: tuple(0 for _ in shape))
    out_sdt = jax.ShapeDtypeStruct((NP, DOUT), F32)
    return pl.pallas_call(
        functools.partial(_dense_body, DH=DH, DOUT=DOUT, NLAYERS=NLAYERS),
        grid=grid,
        in_specs=[
            row_blk(LANES), row_blk(DIN), row_blk(DOUT), row_blk(DWD),
            full(DIN, DH), full(DH, DH), full(NLAYERS, DOUT, DOUT),
            full(DH + DOUT + DWD, DOUT), full(1, DOUT),
        ],
        out_specs=[row_blk(DOUT)] * 5,
        out_shape=[out_sdt] * 5,
    )


def _make_chain_kernel(NP, DOUT, n_chunks, n_fp, n_lp):
    RT = NP // NSUB
    NSC = RT // SUB
    VPR = DOUT // LANES
    ZR = 64

    @functools.partial(
        pl.kernel,
        out_type=[
            jax.ShapeDtypeStruct((NP, DOUT), F32),
            jax.ShapeDtypeStruct((NP, DOUT), F32),
            jax.ShapeDtypeStruct((NP, DOUT), F32),
        ],
        mesh=_sc_mesh(),
        compiler_params=pltpu.CompilerParams(use_tc_tiling_on_sc=False),
        scratch_types=[
            pltpu.VMEM_SHARED((NP, DOUT), F32),
            pltpu.VMEM((n_chunks, CHUNK), jnp.int32),
            pltpu.VMEM((n_chunks, CHUNK), jnp.int32),
            [pltpu.VMEM((CHUNK,), jnp.int32)] * 4,
            pltpu.VMEM((SUB, DOUT), F32),
            pltpu.VMEM((SUB, DOUT), F32),
            pltpu.VMEM((SUB, DOUT), F32),
            pltpu.VMEM((SUB, DOUT), F32),
            pltpu.VMEM((ZR, DOUT), F32),
            pltpu.VMEM((16, DOUT), F32),
            pltpu.SemaphoreType.DMA,
            pltpu.SemaphoreType.DMA,
        ],
    )
    def chain_kernel(src_hbm, dst_hbm, cmat_hbm, ufp0_hbm, ulp0_hbm,
                     zdw_hbm, d1_hbm, d2_hbm, out_hbm, ufp_hbm, ulp_hbm,
                     acc_sh, src_v, dst_v, idxr_v, a_v, u_v, s1_v,
                     s2_v, zero_v, cm_v, gsem, ssem):
        sid = lax.axis_index("s")
        base_r = sid * RT
        base_c = sid * n_chunks

        pltpu.sync_copy(src_hbm.at[pl.ds(base_c, n_chunks)], src_v)
        pltpu.sync_copy(dst_hbm.at[pl.ds(base_c, n_chunks)], dst_v)
        pltpu.sync_copy(cmat_hbm, cm_v)

        def fill_zero(i, _):
            for j in range(VPR):
                zero_v[i, pl.ds(j * LANES, LANES)] = jnp.zeros((LANES,), F32)
            return 0
        lax.fori_loop(0, ZR, fill_zero, 0)

        for i in range(NSC):
            sl = pl.ds(base_r + i * SUB, SUB)
            pltpu.sync_copy(ufp0_hbm.at[sl], u_v)
            pltpu.sync_copy(u_v, ufp_hbm.at[sl])
            pltpu.sync_copy(ulp0_hbm.at[sl], u_v)
            pltpu.sync_copy(u_v, ulp_hbm.at[sl])
        plsc.subcore_barrier()

        def do_step(u_hbm, crow, last):
            for i in range(RT // ZR):
                pltpu.sync_copy(zero_v, acc_sh.at[pl.ds(base_r + i * ZR, ZR)])
            plsc.subcore_barrier()

            bufs = (a_v, u_v, s1_v, s2_v)

            def edge_group(c0, first):
                for b in range(4):
                    if not first:
                        pltpu.make_async_copy(
                            bufs[b], acc_sh.at[idxr_v[b]], ssem).wait()
                    pltpu.async_copy(u_hbm.at[src_v.at[c0 + b]], bufs[b],
                                     gsem)
                for b in range(4):
                    pltpu.make_async_copy(u_hbm.at[src_v.at[c0 + b]],
                                          bufs[b], gsem).wait()
                    for i in range(CHUNK // LANES):
                        idxr_v[b][pl.ds(i * LANES, LANES)] = (
                            dst_v[c0 + b, pl.ds(i * LANES, LANES)])
                    pltpu.async_copy(bufs[b], acc_sh.at[idxr_v[b]],
                                     ssem, add=True)

            edge_group(0, True)

            def edge_loop(g, _):
                edge_group(g * 4, False)
                return 0
            lax.fori_loop(1, n_chunks // 4, edge_loop, 0)
            for b in range(4):
                pltpu.make_async_copy(bufs[b], acc_sh.at[idxr_v[b]],
                                      ssem).wait()
            plsc.subcore_barrier()

            for i in range(NSC):
                sl = pl.ds(base_r + i * SUB, SUB)
                pltpu.sync_copy(acc_sh.at[sl], a_v)
                pltpu.sync_copy(u_hbm.at[sl], u_v)
                if last:
                    pltpu.sync_copy(d1_hbm.at[sl], s2_v)
                else:
                    pltpu.sync_copy(d2_hbm.at[sl], s2_v)
                    pltpu.sync_copy(d1_hbm.at[sl], s1_v)

                def row_body(r, _):
                    for j in range(VPR):
                        cs = pl.ds(j * LANES, LANES)
                        t = (a_v[r, cs] + u_v[r, cs]) * s2_v[r, cs]
                        cv = cm_v[crow, cs]
                        if last:
                            t = t + cv
                        else:
                            t = t + s1_v[r, cs] * cv
                        u_v[r, cs] = t
                    return 0
                lax.fori_loop(0, SUB, row_body, 0)
                pltpu.sync_copy(u_v, u_hbm.at[sl])
            plsc.subcore_barrier()

        for k in range(n_fp - 1):
            do_step(ufp_hbm, k, False)
        do_step(ufp_hbm, n_fp - 1, True)

        def lp_body(k, _):
            do_step(ulp_hbm, n_fp + k, False)
            return 0
        lax.fori_loop(0, n_lp - 1, lp_body, 0)
        do_step(ulp_hbm, n_fp + n_lp - 1, True)

        for i in range(NSC):
            sl = pl.ds(base_r + i * SUB, SUB)
            pltpu.sync_copy(ufp_hbm.at[sl], a_v)
            pltpu.sync_copy(ulp_hbm.at[sl], u_v)
            pltpu.sync_copy(zdw_hbm.at[sl], s1_v)

            def fin_body(r, _):
                for j in range(VPR):
                    cs = pl.ds(j * LANES, LANES)
                    t = a_v[r, cs] + u_v[r, cs] + s1_v[r, cs]
                    a_v[r, cs] = 1.0 / (1.0 + jnp.exp(-t))
                return 0
            lax.fori_loop(0, SUB, fin_body, 0)
            pltpu.sync_copy(a_v, out_hbm.at[sl])

    return chain_kernel


def kernel(x, y, edge_index, deep_walk_emb, gcn_W0, gcn_b0, gcn_W1, gcn_b1,
           label_W, label_b, fus_W, fus_b):
    N, DIN = x.shape
    DOUT = y.shape[1]
    DWD = deep_walk_emb.shape[1]
    DH = gcn_W0.shape[1]
    NLAYERS = label_W.shape[0]
    E = edge_index.shape[1]

    NP = ((N + NSUB * SUB - 1) // (NSUB * SUB)) * (NSUB * SUB)
    n_chunks = (E + NSUB * CHUNK - 1) // (NSUB * CHUNK)
    n_chunks = ((n_chunks + 7) // 8) * 8
    EP = n_chunks * NSUB * CHUNK

    src = edge_index[0]
    dst = edge_index[1]
    pad_e = EP - E
    src_p = jnp.concatenate([src, jnp.zeros((pad_e,), jnp.int32)])
    dst_p = jnp.concatenate([dst, jnp.full((pad_e,), N, jnp.int32)])
    src2 = src_p.reshape(NSUB * n_chunks, CHUNK)
    dst2 = dst_p.reshape(NSUB * n_chunks, CHUNK)

    pad_rows = lambda a: jnp.pad(a, ((0, NP - N), (0, 0)))
    x_p = pad_rows(x)
    y_p = pad_rows(y)
    dw_p = pad_rows(deep_walk_emb)

    Wf_fp = fus_W[:DH]
    Wf_lp = fus_W[DH:DH + DOUT]
    c_fp0 = gcn_b0 @ gcn_W1 @ Wf_fp
    c_fp1 = gcn_b1 @ Wf_fp
    M = Wf_lp
    cs = [None] * NLAYERS
    for k in range(NLAYERS - 1, -1, -1):
        cs[k] = label_b[k] @ M
        M = label_W[k] @ M
    cmat = jnp.zeros((16, DOUT), F32)
    cmat = cmat.at[0].set(c_fp0).at[1].set(c_fp1)
    for k in range(NLAYERS):
        cmat = cmat.at[2 + k].set(cs[k])

    deg_raw = _make_deg_kernel(NP, n_chunks)(dst2)

    ufp0, ulp0, zdwb, d1, d2 = _make_dense_kernel(
        NP, DIN, DH, DOUT, DWD, NLAYERS)(
        deg_raw, x_p, y_p, dw_p, gcn_W0, gcn_W1, label_W, fus_W,
        fus_b.reshape(1, DOUT))

    out, _, _ = _make_chain_kernel(NP, DOUT, n_chunks, 2, NLAYERS)(
        src2, dst2, cmat, ufp0, ulp0, zdwb, d1, d2)

    return out[:N]


# speedup vs baseline: 8.6268x; 1.1942x over previous
import functools

import jax
import jax.numpy as jnp
from jax import lax
from jax.experimental import pallas as pl
from jax.experimental.pallas import tpu as pltpu
from jax.experimental.pallas import tpu_sc as plsc

F32 = jnp.float32
NSUB = 16
LANES = 16
CHUNK = 128
SUB = 128


def _sc_mesh(num_cores=1):
    return plsc.VectorSubcoreMesh(
        core_axis_name="c", subcore_axis_name="s", num_cores=num_cores)


def _make_deg_kernel(NP, n_chunks):
    RT = NP // NSUB

    @functools.partial(
        pl.kernel,
        out_type=jax.ShapeDtypeStruct((NP, LANES), F32),
        mesh=_sc_mesh(),
        compiler_params=pltpu.CompilerParams(use_tc_tiling_on_sc=False),
        scratch_types=[
            pltpu.VMEM_SHARED((NP, LANES), F32),
            pltpu.VMEM((n_chunks, CHUNK), jnp.int32),
            pltpu.VMEM((CHUNK,), jnp.int32),
            pltpu.VMEM((CHUNK, LANES), F32),
            pltpu.VMEM((RT, LANES), F32),
        ],
    )
    def deg_kernel(dst_hbm, deg_out, acc_sh, dst_v, idx_v, ones_v, zero_v):
        sid = lax.axis_index("s")

        def fill_ones(i, _):
            ones_v[i, :] = jnp.ones((LANES,), F32)
            return 0
        lax.fori_loop(0, CHUNK, fill_ones, 0)

        def fill_zero(i, _):
            zero_v[i, :] = jnp.zeros((LANES,), F32)
            return 0
        lax.fori_loop(0, RT, fill_zero, 0)

        pltpu.sync_copy(dst_hbm.at[pl.ds(sid * n_chunks, n_chunks)], dst_v)
        pltpu.sync_copy(zero_v, acc_sh.at[pl.ds(sid * RT, RT)])
        plsc.subcore_barrier()

        def scat(j, _):
            for i in range(CHUNK // LANES):
                idx_v[pl.ds(i * LANES, LANES)] = dst_v[j, pl.ds(i * LANES, LANES)]
            pltpu.sync_copy(ones_v, acc_sh.at[idx_v], add=True)
            return 0
        lax.fori_loop(0, n_chunks, scat, 0)
        plsc.subcore_barrier()

        pltpu.sync_copy(acc_sh.at[pl.ds(sid * RT, RT)],
                        deg_out.at[pl.ds(sid * RT, RT)])

    return deg_kernel


def _dense_body(deg_ref, x_ref, y_ref, dw_ref, w0_ref, w1_ref, lw_ref,
                fw_ref, fb_ref, ufp_ref, ulp_ref, zdw_ref, d1_ref,
                *, DH, DOUT, NLAYERS):
    dot = functools.partial(jnp.dot, precision=lax.Precision.HIGHEST,
                            preferred_element_type=F32)
    deg = deg_ref[...][:, 0:1] + 1.0
    dinv = lax.rsqrt(deg)
    d1_ref[...] = jnp.broadcast_to(dinv, d1_ref.shape)
    fw = fw_ref[...]
    h = dot(x_ref[...], w0_ref[...])
    h = dot(h, w1_ref[...])
    h = dot(h, fw[0:DH])
    ufp_ref[...] = dinv * h
    t = y_ref[...]
    for i in range(NLAYERS):
        t = dot(t, lw_ref[i])
    t = dot(t, fw[DH:DH + DOUT])
    ulp_ref[...] = dinv * t
    zdw_ref[...] = dot(dw_ref[...], fw[DH + DOUT:]) + fb_ref[...]


def _make_dense_kernel(NP, DIN, DH, DOUT, DWD, NLAYERS):
    BR = 512
    grid = (NP // BR,)
    row_blk = lambda w: pl.BlockSpec((BR, w), lambda i: (i, 0))
    full = lambda *shape: pl.BlockSpec(shape, lambda i: tuple(0 for _ in shape))
    out_sdt = jax.ShapeDtypeStruct((NP, DOUT), F32)
    return pl.pallas_call(
        functools.partial(_dense_body, DH=DH, DOUT=DOUT, NLAYERS=NLAYERS),
        grid=grid,
        in_specs=[
            row_blk(LANES), row_blk(DIN), row_blk(DOUT), row_blk(DWD),
            full(DIN, DH), full(DH, DH), full(NLAYERS, DOUT, DOUT),
            full(DH + DOUT + DWD, DOUT), full(1, DOUT),
        ],
        out_specs=[row_blk(DOUT)] * 4,
        out_shape=[out_sdt] * 4,
    )


def _make_chain_kernel(NP, DOUT, n_chunks, n_fp, n_lp):
    RT = NP // NSUB
    NSC = RT // SUB
    VPR = DOUT // LANES
    ZR = 64

    @functools.partial(
        pl.kernel,
        out_type=[
            jax.ShapeDtypeStruct((NP, DOUT), F32),
            jax.ShapeDtypeStruct((NP, DOUT), F32),
        ],
        mesh=_sc_mesh(2),
        compiler_params=pltpu.CompilerParams(use_tc_tiling_on_sc=False),
        scratch_types=[
            pltpu.VMEM_SHARED((NP, DOUT), F32),
            pltpu.VMEM((n_chunks, CHUNK), jnp.int32),
            pltpu.VMEM((n_chunks, CHUNK), jnp.int32),
            [pltpu.VMEM((CHUNK,), jnp.int32)] * 4,
            pltpu.VMEM((SUB, DOUT), F32),
            pltpu.VMEM((SUB, DOUT), F32),
            pltpu.VMEM((SUB, DOUT), F32),
            pltpu.VMEM((SUB, DOUT), F32),
            pltpu.VMEM((ZR, DOUT), F32),
            pltpu.VMEM((16, DOUT), F32),
            pltpu.SemaphoreType.DMA,
            pltpu.SemaphoreType.DMA,
        ],
    )
    def chain_kernel(src_hbm, dst_hbm, cmat_hbm, ufp0_hbm, ulp0_hbm,
                     d1_hbm, ufp_hbm, ulp_hbm,
                     acc_sh, src_v, dst_v, idxr_v, a_v, u_v, s1_v,
                     s2_v, zero_v, cm_v, gsem, ssem):
        cid = lax.axis_index("c")
        sid = lax.axis_index("s")
        base_r = sid * RT
        base_c = sid * n_chunks

        pltpu.sync_copy(src_hbm.at[pl.ds(base_c, n_chunks)], src_v)
        pltpu.sync_copy(dst_hbm.at[pl.ds(base_c, n_chunks)], dst_v)
        pltpu.sync_copy(cmat_hbm, cm_v)

        def fill_zero(i, _):
            for j in range(VPR):
                zero_v[i, pl.ds(j * LANES, LANES)] = jnp.zeros((LANES,), F32)
            return 0
        lax.fori_loop(0, ZR, fill_zero, 0)

        for i in range(NSC):
            sl = pl.ds(base_r + i * SUB, SUB)

            @pl.when(cid == 0)
            def _():
                pltpu.sync_copy(ufp0_hbm.at[sl], u_v)
                pltpu.sync_copy(u_v, ufp_hbm.at[sl])

            @pl.when(cid == 1)
            def _():
                pltpu.sync_copy(ulp0_hbm.at[sl], u_v)
                pltpu.sync_copy(u_v, ulp_hbm.at[sl])
        plsc.subcore_barrier()

        def do_step(u_hbm, crow, last):
            for i in range(RT // ZR):
                pltpu.sync_copy(zero_v, acc_sh.at[pl.ds(base_r + i * ZR, ZR)])
            plsc.subcore_barrier()

            bufs = (a_v, u_v, s1_v, s2_v)

            def edge_group(c0, first):
                for b in range(4):
                    if not first:
                        pltpu.make_async_copy(
                            bufs[b], acc_sh.at[idxr_v[b]], ssem).wait()
                    pltpu.async_copy(u_hbm.at[src_v.at[c0 + b]], bufs[b],
                                     gsem)
                for b in range(4):
                    pltpu.make_async_copy(u_hbm.at[src_v.at[c0 + b]],
                                          bufs[b], gsem).wait()
                    for i in range(CHUNK // LANES):
                        idxr_v[b][pl.ds(i * LANES, LANES)] = (
                            dst_v[c0 + b, pl.ds(i * LANES, LANES)])
                    pltpu.async_copy(bufs[b], acc_sh.at[idxr_v[b]],
                                     ssem, add=True)

            edge_group(0, True)

            def edge_loop(g, _):
                edge_group(g * 4, False)
                return 0
            lax.fori_loop(1, n_chunks // 4, edge_loop, 0)
            for b in range(4):
                pltpu.make_async_copy(bufs[b], acc_sh.at[idxr_v[b]],
                                      ssem).wait()
            plsc.subcore_barrier()

            for i in range(NSC):
                sl = pl.ds(base_r + i * SUB, SUB)
                pltpu.sync_copy(acc_sh.at[sl], a_v)
                pltpu.sync_copy(u_hbm.at[sl], u_v)
                pltpu.sync_copy(d1_hbm.at[sl], s2_v)

                def row_body(r, _):
                    for j in range(VPR):
                        cs = pl.ds(j * LANES, LANES)
                        d = s2_v[r, cs]
                        t = d * (a_v[r, cs] + u_v[r, cs]) + cm_v[crow, cs]
                        if not last:
                            t = t * d
                        u_v[r, cs] = t
                    return 0
                lax.fori_loop(0, SUB, row_body, 0)
                pltpu.sync_copy(u_v, u_hbm.at[sl])
            plsc.subcore_barrier()

        @pl.when(cid == 0)
        def _():
            for k in range(n_fp - 1):
                do_step(ufp_hbm, k, False)
            do_step(ufp_hbm, n_fp - 1, True)

        @pl.when(cid == 1)
        def _():
            def lp_body(k, _):
                do_step(ulp_hbm, n_fp + k, False)
                return 0
            lax.fori_loop(0, n_lp - 1, lp_body, 0)
            do_step(ulp_hbm, n_fp + n_lp - 1, True)

    return chain_kernel


def _head_body(vfp_ref, vlp_ref, zdw_ref, out_ref):
    out_ref[...] = jax.nn.sigmoid(vfp_ref[...] + vlp_ref[...] + zdw_ref[...])


def _make_head_kernel(NP, DOUT):
    BR = 512
    blk = pl.BlockSpec((BR, DOUT), lambda i: (i, 0))
    return pl.pallas_call(
        _head_body,
        grid=(NP // BR,),
        in_specs=[blk, blk, blk],
        out_specs=blk,
        out_shape=jax.ShapeDtypeStruct((NP, DOUT), F32),
    )


def kernel(x, y, edge_index, deep_walk_emb, gcn_W0, gcn_b0, gcn_W1, gcn_b1,
           label_W, label_b, fus_W, fus_b):
    N, DIN = x.shape
    DOUT = y.shape[1]
    DWD = deep_walk_emb.shape[1]
    DH = gcn_W0.shape[1]
    NLAYERS = label_W.shape[0]
    E = edge_index.shape[1]

    NP = ((N + NSUB * SUB - 1) // (NSUB * SUB)) * (NSUB * SUB)
    n_chunks = (E + NSUB * CHUNK - 1) // (NSUB * CHUNK)
    n_chunks = ((n_chunks + 7) // 8) * 8
    EP = n_chunks * NSUB * CHUNK

    src = edge_index[0]
    dst = edge_index[1]
    pad_e = EP - E
    src_p = jnp.concatenate([src, jnp.zeros((pad_e,), jnp.int32)])
    dst_p = jnp.concatenate([dst, jnp.full((pad_e,), N, jnp.int32)])
    src2 = src_p.reshape(NSUB * n_chunks, CHUNK)
    dst2 = dst_p.reshape(NSUB * n_chunks, CHUNK)

    pad_rows = lambda a: jnp.pad(a, ((0, NP - N), (0, 0)))
    x_p = pad_rows(x)
    y_p = pad_rows(y)
    dw_p = pad_rows(deep_walk_emb)

    Wf_fp = fus_W[:DH]
    Wf_lp = fus_W[DH:DH + DOUT]
    c_fp0 = gcn_b0 @ gcn_W1 @ Wf_fp
    c_fp1 = gcn_b1 @ Wf_fp
    M = Wf_lp
    cs = [None] * NLAYERS
    for k in range(NLAYERS - 1, -1, -1):
        cs[k] = label_b[k] @ M
        M = label_W[k] @ M
    cmat = jnp.zeros((16, DOUT), F32)
    cmat = cmat.at[0].set(c_fp0).at[1].set(c_fp1)
    for k in range(NLAYERS):
        cmat = cmat.at[2 + k].set(cs[k])

    deg_raw = _make_deg_kernel(NP, n_chunks)(dst2)

    ufp0, ulp0, zdwb, d1 = _make_dense_kernel(
        NP, DIN, DH, DOUT, DWD, NLAYERS)(
        deg_raw, x_p, y_p, dw_p, gcn_W0, gcn_W1, label_W, fus_W,
        fus_b.reshape(1, DOUT))

    vfp, vlp = _make_chain_kernel(NP, DOUT, n_chunks, 2, NLAYERS)(
        src2, dst2, cmat, ufp0, ulp0, d1)

    out = _make_head_kernel(NP, DOUT)(vfp, vlp, zdwb)

    return out[:N]


# speedup vs baseline: 9.8723x; 1.1444x over previous
import functools

import jax
import jax.numpy as jnp
from jax import lax
from jax.experimental import pallas as pl
from jax.experimental.pallas import tpu as pltpu
from jax.experimental.pallas import tpu_sc as plsc

F32 = jnp.float32
NSUB = 16
LANES = 16
CHUNK = 128
SUB = 128


def _sc_mesh(num_cores=1):
    return plsc.VectorSubcoreMesh(
        core_axis_name="c", subcore_axis_name="s", num_cores=num_cores)


def _make_deg_kernel(NP, n_chunks):
    RT = NP // NSUB

    @functools.partial(
        pl.kernel,
        out_type=jax.ShapeDtypeStruct((NP, LANES), F32),
        mesh=_sc_mesh(),
        compiler_params=pltpu.CompilerParams(use_tc_tiling_on_sc=False),
        scratch_types=[
            pltpu.VMEM_SHARED((NP, LANES), F32),
            pltpu.VMEM((n_chunks, CHUNK), jnp.int32),
            pltpu.VMEM((CHUNK,), jnp.int32),
            pltpu.VMEM((CHUNK, LANES), F32),
            pltpu.VMEM((RT, LANES), F32),
        ],
    )
    def deg_kernel(dst_hbm, deg_out, acc_sh, dst_v, idx_v, ones_v, zero_v):
        sid = lax.axis_index("s")

        def fill_ones(i, _):
            ones_v[i, :] = jnp.ones((LANES,), F32)
            return 0
        lax.fori_loop(0, CHUNK, fill_ones, 0)

        def fill_zero(i, _):
            zero_v[i, :] = jnp.zeros((LANES,), F32)
            return 0
        lax.fori_loop(0, RT, fill_zero, 0)

        pltpu.sync_copy(dst_hbm.at[pl.ds(sid * n_chunks, n_chunks)], dst_v)
        pltpu.sync_copy(zero_v, acc_sh.at[pl.ds(sid * RT, RT)])
        plsc.subcore_barrier()

        def scat(j, _):
            for i in range(CHUNK // LANES):
                idx_v[pl.ds(i * LANES, LANES)] = dst_v[j, pl.ds(i * LANES, LANES)]
            pltpu.sync_copy(ones_v, acc_sh.at[idx_v], add=True)
            return 0
        lax.fori_loop(0, n_chunks, scat, 0)
        plsc.subcore_barrier()

        pltpu.sync_copy(acc_sh.at[pl.ds(sid * RT, RT)],
                        deg_out.at[pl.ds(sid * RT, RT)])

    return deg_kernel


def _dense_body(deg_ref, x_ref, y_ref, dw_ref, w0_ref, w1_ref, lw_ref,
                fw_ref, fb_ref, ufp_ref, ulp_ref, zdw_ref, d1_ref,
                *, DH, DOUT, NLAYERS):
    dot = functools.partial(jnp.dot, precision=lax.Precision.HIGHEST,
                            preferred_element_type=F32)
    deg = deg_ref[...][:, 0:1] + 1.0
    dinv = lax.rsqrt(deg)
    d1_ref[...] = jnp.broadcast_to(dinv, d1_ref.shape)
    fw = fw_ref[...]
    h = dot(x_ref[...], w0_ref[...])
    h = dot(h, w1_ref[...])
    h = dot(h, fw[0:DH])
    ufp_ref[...] = dinv * h
    t = y_ref[...]
    for i in range(NLAYERS):
        t = dot(t, lw_ref[i])
    t = dot(t, fw[DH:DH + DOUT])
    ulp_ref[...] = dinv * t
    zdw_ref[...] = dot(dw_ref[...], fw[DH + DOUT:]) + fb_ref[...]


def _make_dense_kernel(NP, DIN, DH, DOUT, DWD, NLAYERS):
    BR = 512
    grid = (NP // BR,)
    row_blk = lambda w: pl.BlockSpec((BR, w), lambda i: (i, 0))
    full = lambda *shape: pl.BlockSpec(shape, lambda i: tuple(0 for _ in shape))
    out_sdt = jax.ShapeDtypeStruct((NP, DOUT), F32)
    return pl.pallas_call(
        functools.partial(_dense_body, DH=DH, DOUT=DOUT, NLAYERS=NLAYERS),
        grid=grid,
        in_specs=[
            row_blk(LANES), row_blk(DIN), row_blk(DOUT), row_blk(DWD),
            full(DIN, DH), full(DH, DH), full(NLAYERS, DOUT, DOUT),
            full(DH + DOUT + DWD, DOUT), full(1, DOUT),
        ],
        out_specs=[row_blk(DOUT)] * 4,
        out_shape=[out_sdt] * 4,
    )


def _make_chain_kernel(NP, DOUT, n_chunks, n_fp, n_lp):
    NW = 2 * NSUB
    RT = NP // NSUB
    RW = NP // NW
    CSUB = 64
    NCS = RW // CSUB
    NCH = n_chunks
    VPR = DOUT // LANES
    ZR = 64

    @functools.partial(
        pl.kernel,
        out_type=[
            jax.ShapeDtypeStruct((NP, DOUT), F32),
            jax.ShapeDtypeStruct((NP, DOUT), F32),
            jax.ShapeDtypeStruct((NP, DOUT), F32),
            jax.ShapeDtypeStruct((NP, DOUT), F32),
        ],
        mesh=_sc_mesh(2),
        compiler_params=pltpu.CompilerParams(use_tc_tiling_on_sc=False),
        scratch_types=[
            pltpu.VMEM_SHARED((NP, DOUT), F32),
            pltpu.VMEM((n_chunks, CHUNK), jnp.int32),
            pltpu.VMEM((n_chunks, CHUNK), jnp.int32),
            [pltpu.VMEM((CHUNK,), jnp.int32)] * 4,
            pltpu.VMEM((SUB, DOUT), F32),
            pltpu.VMEM((SUB, DOUT), F32),
            pltpu.VMEM((SUB, DOUT), F32),
            pltpu.VMEM((SUB, DOUT), F32),
            pltpu.VMEM((ZR, DOUT), F32),
            pltpu.VMEM((16, DOUT), F32),
            pltpu.SemaphoreType.DMA,
            pltpu.SemaphoreType.DMA,
            pltpu.SemaphoreType.REGULAR,
        ],
    )
    def chain_kernel(src_hbm, dst_hbm, cmat_hbm, ufp0_hbm, ulp0_hbm,
                     d1_hbm, ufp_hbm, ulp_hbm, p0_hbm, p1_hbm,
                     acc_sh, src_v, dst_v, idxr_v, p0_v, p1_v, u_v,
                     d_v, zero_v, cm_v, gsem, ssem, xsem):
        cid = lax.axis_index("c")
        sid = lax.axis_index("s")
        wid = sid * 2 + cid
        base_r = sid * RT
        base_w = wid * RW
        base_c = wid * n_chunks
        a_v, s2_v = p0_v, d_v

        def xbarrier():
            plsc.subcore_barrier()

            @pl.when(sid == 0)
            def _():
                pl.semaphore_signal(xsem, 1, core_index=1 - cid)
                pl.semaphore_wait(xsem, 1)
            plsc.subcore_barrier()

        pltpu.sync_copy(src_hbm.at[pl.ds(base_c, n_chunks)], src_v)
        pltpu.sync_copy(dst_hbm.at[pl.ds(base_c, n_chunks)], dst_v)
        pltpu.sync_copy(cmat_hbm, cm_v)

        def fill_zero(i, _):
            for j in range(VPR):
                zero_v[i, pl.ds(j * LANES, LANES)] = jnp.zeros((LANES,), F32)
            return 0
        lax.fori_loop(0, ZR, fill_zero, 0)

        for i in range(NCS):
            sl = pl.ds(base_w + i * CSUB, CSUB)
            pltpu.sync_copy(ufp0_hbm.at[sl], u_v.at[pl.ds(0, CSUB)])
            pltpu.sync_copy(u_v.at[pl.ds(0, CSUB)], ufp_hbm.at[sl])
            pltpu.sync_copy(ulp0_hbm.at[sl], u_v.at[pl.ds(0, CSUB)])
            pltpu.sync_copy(u_v.at[pl.ds(0, CSUB)], ulp_hbm.at[sl])
        xbarrier()

        def do_step(u_hbm, crow, last):
            for i in range(RT // ZR):
                pltpu.sync_copy(zero_v, acc_sh.at[pl.ds(base_r + i * ZR, ZR)])
            plsc.subcore_barrier()

            bufs = (p0_v, p1_v, u_v, d_v)

            def edge_group(c0, first):
                for b in range(4):
                    if not first:
                        pltpu.make_async_copy(
                            bufs[b], acc_sh.at[idxr_v[b]], ssem).wait()
                    pltpu.async_copy(u_hbm.at[src_v.at[c0 + b]], bufs[b],
                                     gsem)
                for b in range(4):
                    pltpu.make_async_copy(u_hbm.at[src_v.at[c0 + b]],
                                          bufs[b], gsem).wait()
                    for i in range(CHUNK // LANES):
                        idxr_v[b][pl.ds(i * LANES, LANES)] = (
                            dst_v[c0 + b, pl.ds(i * LANES, LANES)])
                    pltpu.async_copy(bufs[b], acc_sh.at[idxr_v[b]],
                                     ssem, add=True)

            edge_group(0, True)

            def edge_loop(g, _):
                edge_group(g * 4, False)
                return 0
            lax.fori_loop(1, n_chunks // 4, edge_loop, 0)
            for b in range(4):
                pltpu.make_async_copy(bufs[b], acc_sh.at[idxr_v[b]],
                                      ssem).wait()
            plsc.subcore_barrier()

            sl_t = pl.ds(base_r, RT)

            @pl.when(cid == 0)
            def _():
                pltpu.sync_copy(acc_sh.at[sl_t], p0_hbm.at[sl_t])

            @pl.when(cid == 1)
            def _():
                pltpu.sync_copy(acc_sh.at[sl_t], p1_hbm.at[sl_t])
            xbarrier()

            for i in range(NCS):
                sl = pl.ds(base_w + i * CSUB, CSUB)
                cs0 = pl.ds(0, CSUB)
                pltpu.sync_copy(p0_hbm.at[sl], p0_v.at[cs0])
                pltpu.sync_copy(p1_hbm.at[sl], p1_v.at[cs0])
                pltpu.sync_copy(u_hbm.at[sl], u_v.at[cs0])
                pltpu.sync_copy(d1_hbm.at[sl], d_v.at[cs0])

                def row_body(r, _):
                    for j in range(VPR):
                        cs = pl.ds(j * LANES, LANES)
                        d = d_v[r, cs]
                        t = d * (p0_v[r, cs] + p1_v[r, cs] + u_v[r, cs])
                        t = t + cm_v[crow, cs]
                        if not last:
                            t = t * d
                        u_v[r, cs] = t
                    return 0
                lax.fori_loop(0, CSUB, row_body, 0)
                pltpu.sync_copy(u_v.at[cs0], u_hbm.at[sl])
            xbarrier()

        for k in range(n_fp - 1):
            do_step(ufp_hbm, k, False)
        do_step(ufp_hbm, n_fp - 1, True)

        def lp_body(k, _):
            do_step(ulp_hbm, n_fp + k, False)
            return 0
        lax.fori_loop(0, n_lp - 1, lp_body, 0)
        do_step(ulp_hbm, n_fp + n_lp - 1, True)

    return chain_kernel


def _head_body(vfp_ref, vlp_ref, zdw_ref, out_ref):
    out_ref[...] = jax.nn.sigmoid(vfp_ref[...] + vlp_ref[...] + zdw_ref[...])


def _make_head_kernel(NP, DOUT):
    BR = 512
    blk = pl.BlockSpec((BR, DOUT), lambda i: (i, 0))
    return pl.pallas_call(
        _head_body,
        grid=(NP // BR,),
        in_specs=[blk, blk, blk],
        out_specs=blk,
        out_shape=jax.ShapeDtypeStruct((NP, DOUT), F32),
    )


def kernel(x, y, edge_index, deep_walk_emb, gcn_W0, gcn_b0, gcn_W1, gcn_b1,
           label_W, label_b, fus_W, fus_b):
    N, DIN = x.shape
    DOUT = y.shape[1]
    DWD = deep_walk_emb.shape[1]
    DH = gcn_W0.shape[1]
    NLAYERS = label_W.shape[0]
    E = edge_index.shape[1]

    NW = 2 * NSUB
    NP = ((N + NW * 64 - 1) // (NW * 64)) * (NW * 64)
    n_chunks = (E + NW * CHUNK - 1) // (NW * CHUNK)
    n_chunks = ((n_chunks + 7) // 8) * 8
    EP = n_chunks * NW * CHUNK

    src = edge_index[0]
    dst = edge_index[1]
    pad_e = EP - E
    src_p = jnp.concatenate([src, jnp.zeros((pad_e,), jnp.int32)])
    dst_p = jnp.concatenate([dst, jnp.full((pad_e,), N, jnp.int32)])
    src2 = src_p.reshape(NW * n_chunks, CHUNK)
    dst2 = dst_p.reshape(NW * n_chunks, CHUNK)

    pad_rows = lambda a: jnp.pad(a, ((0, NP - N), (0, 0)))
    x_p = pad_rows(x)
    y_p = pad_rows(y)
    dw_p = pad_rows(deep_walk_emb)

    Wf_fp = fus_W[:DH]
    Wf_lp = fus_W[DH:DH + DOUT]
    c_fp0 = gcn_b0 @ gcn_W1 @ Wf_fp
    c_fp1 = gcn_b1 @ Wf_fp
    M = Wf_lp
    cs = [None] * NLAYERS
    for k in range(NLAYERS - 1, -1, -1):
        cs[k] = label_b[k] @ M
        M = label_W[k] @ M
    cmat = jnp.zeros((16, DOUT), F32)
    cmat = cmat.at[0].set(c_fp0).at[1].set(c_fp1)
    for k in range(NLAYERS):
        cmat = cmat.at[2 + k].set(cs[k])

    deg_raw = _make_deg_kernel(NP, 2 * n_chunks)(dst2)

    ufp0, ulp0, zdwb, d1 = _make_dense_kernel(
        NP, DIN, DH, DOUT, DWD, NLAYERS)(
        deg_raw, x_p, y_p, dw_p, gcn_W0, gcn_W1, label_W, fus_W,
        fus_b.reshape(1, DOUT))

    vfp, vlp, _, _ = _make_chain_kernel(NP, DOUT, n_chunks, 2, NLAYERS)(
        src2, dst2, cmat, ufp0, ulp0, d1)

    out = _make_head_kernel(NP, DOUT)(vfp, vlp, zdwb)

    return out[:N]


# speedup vs baseline: 10.0427x; 1.0173x over previous
import functools

import jax
import jax.numpy as jnp
from jax import lax
from jax.experimental import pallas as pl
from jax.experimental.pallas import tpu as pltpu
from jax.experimental.pallas import tpu_sc as plsc

F32 = jnp.float32
NSUB = 16
LANES = 16
CHUNK = 128
SUB = 128


def _sc_mesh(num_cores=1):
    return plsc.VectorSubcoreMesh(
        core_axis_name="c", subcore_axis_name="s", num_cores=num_cores)


def _make_deg_kernel(NP, n_chunks):
    RT = NP // NSUB

    @functools.partial(
        pl.kernel,
        out_type=jax.ShapeDtypeStruct((NP, LANES), F32),
        mesh=_sc_mesh(),
        compiler_params=pltpu.CompilerParams(use_tc_tiling_on_sc=False),
        scratch_types=[
            pltpu.VMEM_SHARED((NP, LANES), F32),
            pltpu.VMEM((n_chunks, CHUNK), jnp.int32),
            pltpu.VMEM((CHUNK,), jnp.int32),
            pltpu.VMEM((CHUNK, LANES), F32),
            pltpu.VMEM((RT, LANES), F32),
        ],
    )
    def deg_kernel(dst_hbm, deg_out, acc_sh, dst_v, idx_v, ones_v, zero_v):
        sid = lax.axis_index("s")

        def fill_ones(i, _):
            ones_v[i, :] = jnp.ones((LANES,), F32)
            return 0
        lax.fori_loop(0, CHUNK, fill_ones, 0)

        def fill_zero(i, _):
            zero_v[i, :] = jnp.zeros((LANES,), F32)
            return 0
        lax.fori_loop(0, RT, fill_zero, 0)

        pltpu.sync_copy(dst_hbm.at[pl.ds(sid * n_chunks, n_chunks)], dst_v)
        pltpu.sync_copy(zero_v, acc_sh.at[pl.ds(sid * RT, RT)])
        plsc.subcore_barrier()

        def scat(j, _):
            for i in range(CHUNK // LANES):
                idx_v[pl.ds(i * LANES, LANES)] = dst_v[j, pl.ds(i * LANES, LANES)]
            pltpu.sync_copy(ones_v, acc_sh.at[idx_v], add=True)
            return 0
        lax.fori_loop(0, n_chunks, scat, 0)
        plsc.subcore_barrier()

        pltpu.sync_copy(acc_sh.at[pl.ds(sid * RT, RT)],
                        deg_out.at[pl.ds(sid * RT, RT)])

    return deg_kernel


def _dense_body(deg_ref, x_ref, y_ref, dw_ref, w0_ref, w1_ref, lw_ref,
                fw_ref, fb_ref, ufp_ref, ulp_ref, zdw_ref, d1_ref,
                *, DH, DOUT, NLAYERS):
    dot = functools.partial(jnp.dot, precision=lax.Precision.HIGHEST,
                            preferred_element_type=F32)
    deg = deg_ref[...][:, 0:1] + 1.0
    dinv = lax.rsqrt(deg)
    d1_ref[...] = jnp.broadcast_to(dinv, d1_ref.shape)
    fw = fw_ref[...]
    h = dot(x_ref[...], w0_ref[...])
    h = dot(h, w1_ref[...])
    h = dot(h, fw[0:DH])
    ufp_ref[...] = dinv * h
    t = y_ref[...]
    for i in range(NLAYERS):
        t = dot(t, lw_ref[i])
    t = dot(t, fw[DH:DH + DOUT])
    ulp_ref[...] = dinv * t
    zdw_ref[...] = dot(dw_ref[...], fw[DH + DOUT:]) + fb_ref[...]


def _make_dense_kernel(NP, DIN, DH, DOUT, DWD, NLAYERS):
    BR = 512
    grid = (NP // BR,)
    row_blk = lambda w: pl.BlockSpec((BR, w), lambda i: (i, 0))
    full = lambda *shape: pl.BlockSpec(shape, lambda i: tuple(0 for _ in shape))
    out_sdt = jax.ShapeDtypeStruct((NP, DOUT), F32)
    return pl.pallas_call(
        functools.partial(_dense_body, DH=DH, DOUT=DOUT, NLAYERS=NLAYERS),
        grid=grid,
        in_specs=[
            row_blk(LANES), row_blk(DIN), row_blk(DOUT), row_blk(DWD),
            full(DIN, DH), full(DH, DH), full(NLAYERS, DOUT, DOUT),
            full(DH + DOUT + DWD, DOUT), full(1, DOUT),
        ],
        out_specs=[row_blk(DOUT)] * 4,
        out_shape=[out_sdt] * 4,
    )


def _make_chain_kernel(NP, DOUT, n_chunks, n_fp, n_lp):
    NW = 2 * NSUB
    RT = NP // NSUB
    RW = NP // NW
    CSUB = 64
    NCS = RW // CSUB
    NCH = n_chunks
    VPR = DOUT // LANES
    ZR = 64

    @functools.partial(
        pl.kernel,
        out_type=[
            jax.ShapeDtypeStruct((NP, DOUT), F32),
            jax.ShapeDtypeStruct((NP, DOUT), F32),
            jax.ShapeDtypeStruct((NP, DOUT), F32),
            jax.ShapeDtypeStruct((NP, DOUT), F32),
        ],
        mesh=_sc_mesh(2),
        compiler_params=pltpu.CompilerParams(use_tc_tiling_on_sc=False),
        scratch_types=[
            pltpu.VMEM_SHARED((NP, DOUT), F32),
            pltpu.VMEM((n_chunks, CHUNK), jnp.int32),
            pltpu.VMEM((n_chunks, CHUNK), jnp.int32),
            [pltpu.VMEM((CHUNK,), jnp.int32)] * 4,
            pltpu.VMEM((SUB, DOUT), F32),
            pltpu.VMEM((SUB, DOUT), F32),
            pltpu.VMEM((SUB, DOUT), F32),
            pltpu.VMEM((SUB, DOUT), F32),
            pltpu.VMEM((ZR, DOUT), F32),
            pltpu.VMEM((16, DOUT), F32),
            pltpu.SemaphoreType.DMA,
            pltpu.SemaphoreType.DMA,
            pltpu.SemaphoreType.REGULAR,
        ],
    )
    def chain_kernel(src_hbm, dst_hbm, cmat_hbm, ufp0_hbm, ulp0_hbm,
                     d1_hbm, ufp_hbm, ulp_hbm, p0_hbm, p1_hbm,
                     acc_sh, src_v, dst_v, idxr_v, p0_v, p1_v, u_v,
                     d_v, zero_v, cm_v, gsem, ssem, xsem):
        cid = lax.axis_index("c")
        sid = lax.axis_index("s")
        wid = sid * 2 + cid
        base_r = sid * RT
        base_w = wid * RW
        base_c = wid * n_chunks
        a_v, s2_v = p0_v, d_v

        def xbarrier():
            plsc.subcore_barrier()

            @pl.when(sid == 0)
            def _():
                pl.semaphore_signal(xsem, 1, core_index=1 - cid)
                pl.semaphore_wait(xsem, 1)
            plsc.subcore_barrier()

        pltpu.sync_copy(src_hbm.at[pl.ds(base_c, n_chunks)], src_v)
        pltpu.sync_copy(dst_hbm.at[pl.ds(base_c, n_chunks)], dst_v)
        pltpu.sync_copy(cmat_hbm, cm_v)

        def fill_zero(i, _):
            for j in range(VPR):
                zero_v[i, pl.ds(j * LANES, LANES)] = jnp.zeros((LANES,), F32)
            return 0
        lax.fori_loop(0, ZR, fill_zero, 0)

        for i in range(NCS):
            sl = pl.ds(base_w + i * CSUB, CSUB)
            pltpu.sync_copy(ufp0_hbm.at[sl], u_v.at[pl.ds(0, CSUB)])
            pltpu.sync_copy(u_v.at[pl.ds(0, CSUB)], ufp_hbm.at[sl])
            pltpu.sync_copy(ulp0_hbm.at[sl], u_v.at[pl.ds(0, CSUB)])
            pltpu.sync_copy(u_v.at[pl.ds(0, CSUB)], ulp_hbm.at[sl])
        xbarrier()

        for i in range(RT // ZR):
            pltpu.sync_copy(zero_v, acc_sh.at[pl.ds(base_r + i * ZR, ZR)])
        plsc.subcore_barrier()

        def do_step(u_hbm, crow, last):

            bufs = (p0_v, p1_v, u_v, d_v)

            def edge_group(c0, first):
                for b in range(4):
                    if not first:
                        pltpu.make_async_copy(
                            bufs[b], acc_sh.at[idxr_v[b]], ssem).wait()
                    pltpu.async_copy(u_hbm.at[src_v.at[c0 + b]], bufs[b],
                                     gsem)
                for b in range(4):
                    pltpu.make_async_copy(u_hbm.at[src_v.at[c0 + b]],
                                          bufs[b], gsem).wait()
                    for i in range(CHUNK // LANES):
                        idxr_v[b][pl.ds(i * LANES, LANES)] = (
                            dst_v[c0 + b, pl.ds(i * LANES, LANES)])
                    pltpu.async_copy(bufs[b], acc_sh.at[idxr_v[b]],
                                     ssem, add=True)

            edge_group(0, True)

            def edge_loop(g, _):
                edge_group(g * 4, False)
                return 0
            lax.fori_loop(1, n_chunks // 4, edge_loop, 0)
            for b in range(4):
                pltpu.make_async_copy(bufs[b], acc_sh.at[idxr_v[b]],
                                      ssem).wait()
            plsc.subcore_barrier()

            sl_t = pl.ds(base_r, RT)

            @pl.when(cid == 0)
            def _():
                pltpu.sync_copy(acc_sh.at[sl_t], p0_hbm.at[sl_t])

            @pl.when(cid == 1)
            def _():
                pltpu.sync_copy(acc_sh.at[sl_t], p1_hbm.at[sl_t])
            for i in range(RT // ZR):
                pltpu.sync_copy(zero_v, acc_sh.at[pl.ds(base_r + i * ZR, ZR)])
            xbarrier()

            for i in range(NCS):
                sl = pl.ds(base_w + i * CSUB, CSUB)
                cs0 = pl.ds(0, CSUB)
                pltpu.sync_copy(p0_hbm.at[sl], p0_v.at[cs0])
                pltpu.sync_copy(p1_hbm.at[sl], p1_v.at[cs0])
                pltpu.sync_copy(u_hbm.at[sl], u_v.at[cs0])
                pltpu.sync_copy(d1_hbm.at[sl], d_v.at[cs0])

                cvs = [cm_v[crow, pl.ds(j * LANES, LANES)]
                       for j in range(VPR)]

                def row_body(r, _):
                    for j in range(VPR):
                        cs = pl.ds(j * LANES, LANES)
                        d = d_v[r, cs]
                        t = d * (p0_v[r, cs] + p1_v[r, cs] + u_v[r, cs])
                        t = t + cvs[j]
                        if not last:
                            t = t * d
                        u_v[r, cs] = t
                    return 0
                lax.fori_loop(0, CSUB, row_body, 0)
                pltpu.sync_copy(u_v.at[cs0], u_hbm.at[sl])
            xbarrier()

        for k in range(n_fp - 1):
            do_step(ufp_hbm, k, False)
        do_step(ufp_hbm, n_fp - 1, True)

        def lp_body(k, _):
            do_step(ulp_hbm, n_fp + k, False)
            return 0
        lax.fori_loop(0, n_lp - 1, lp_body, 0)
        do_step(ulp_hbm, n_fp + n_lp - 1, True)

    return chain_kernel


def _head_body(vfp_ref, vlp_ref, zdw_ref, out_ref):
    out_ref[...] = jax.nn.sigmoid(vfp_ref[...] + vlp_ref[...] + zdw_ref[...])


def _make_head_kernel(NP, DOUT):
    BR = 512
    blk = pl.BlockSpec((BR, DOUT), lambda i: (i, 0))
    return pl.pallas_call(
        _head_body,
        grid=(NP // BR,),
        in_specs=[blk, blk, blk],
        out_specs=blk,
        out_shape=jax.ShapeDtypeStruct((NP, DOUT), F32),
    )


def kernel(x, y, edge_index, deep_walk_emb, gcn_W0, gcn_b0, gcn_W1, gcn_b1,
           label_W, label_b, fus_W, fus_b):
    N, DIN = x.shape
    DOUT = y.shape[1]
    DWD = deep_walk_emb.shape[1]
    DH = gcn_W0.shape[1]
    NLAYERS = label_W.shape[0]
    E = edge_index.shape[1]

    NW = 2 * NSUB
    NP = ((N + NW * 64 - 1) // (NW * 64)) * (NW * 64)
    n_chunks = (E + NW * CHUNK - 1) // (NW * CHUNK)
    n_chunks = ((n_chunks + 7) // 8) * 8
    EP = n_chunks * NW * CHUNK

    src = edge_index[0]
    dst = edge_index[1]
    pad_e = EP - E
    src_p = jnp.concatenate([src, jnp.zeros((pad_e,), jnp.int32)])
    dst_p = jnp.concatenate([dst, jnp.full((pad_e,), N, jnp.int32)])
    src2 = src_p.reshape(NW * n_chunks, CHUNK)
    dst2 = dst_p.reshape(NW * n_chunks, CHUNK)

    pad_rows = lambda a: jnp.pad(a, ((0, NP - N), (0, 0)))
    x_p = pad_rows(x)
    y_p = pad_rows(y)
    dw_p = pad_rows(deep_walk_emb)

    Wf_fp = fus_W[:DH]
    Wf_lp = fus_W[DH:DH + DOUT]
    c_fp0 = gcn_b0 @ gcn_W1 @ Wf_fp
    c_fp1 = gcn_b1 @ Wf_fp
    M = Wf_lp
    cs = [None] * NLAYERS
    for k in range(NLAYERS - 1, -1, -1):
        cs[k] = label_b[k] @ M
        M = label_W[k] @ M
    cmat = jnp.zeros((16, DOUT), F32)
    cmat = cmat.at[0].set(c_fp0).at[1].set(c_fp1)
    for k in range(NLAYERS):
        cmat = cmat.at[2 + k].set(cs[k])

    deg_raw = _make_deg_kernel(NP, 2 * n_chunks)(dst2)

    ufp0, ulp0, zdwb, d1 = _make_dense_kernel(
        NP, DIN, DH, DOUT, DWD, NLAYERS)(
        deg_raw, x_p, y_p, dw_p, gcn_W0, gcn_W1, label_W, fus_W,
        fus_b.reshape(1, DOUT))

    vfp, vlp, _, _ = _make_chain_kernel(NP, DOUT, n_chunks, 2, NLAYERS)(
        src2, dst2, cmat, ufp0, ulp0, d1)

    out = _make_head_kernel(NP, DOUT)(vfp, vlp, zdwb)

    return out[:N]
